# baseline (speedup 1.0000x reference)
import functools

import jax
import jax.numpy as jnp
from jax.experimental import pallas as pl
from jax.experimental.pallas import tpu as pltpu

_TT = 128
_NBUF = 4


def _stripe_copy(h_hbm, hbuf, sems, j, slot, *, tt):
    return pltpu.make_async_copy(
        h_hbm.at[:, pl.ds(j * tt, tt)],
        hbuf.at[slot],
        sems.at[slot],
    )


def _acc_kernel(h_hbm, xa_ref, out_ref, s_ref, hbuf, acc_ref, sacc_ref, sems,
                *, nsteps, tag_num, tt):
    k = pl.program_id(0)
    slot = jax.lax.rem(k, _NBUF)
    cp = functools.partial(_stripe_copy, h_hbm, hbuf, sems, tt=tt)

    @pl.when(k == 0)
    def _prime():
        for j in range(_NBUF):
            cp(j, j).start()

    cp(k, slot).wait()

    h = hbuf[slot]

    lane = jax.lax.broadcasted_iota(jnp.int32, h.shape, 1)
    hb = jnp.where(lane < (tag_num - k * tt), h.astype(jnp.bfloat16), 0)

    m = jax.lax.dot_general(
        hb, xa_ref[...], (((0,), (0,)), ((), ())),
        preferred_element_type=jnp.float32,
    )
    b = m[:, 128:129]
    b_inv = 1.0 / jnp.where(b == 0.0, 1.0, b)
    mp = (m[:, :128] * b_inv).astype(jnp.bfloat16)

    outp = jnp.dot(hb, mp, preferred_element_type=jnp.float32)

    @pl.when(k == 0)
    def _init():
        acc_ref[...] = outp
        sacc_ref[...] = hb

    @pl.when(k != 0)
    def _acc():
        acc_ref[...] += outp
        sacc_ref[...] += hb

    jj = k + _NBUF

    @pl.when(jj < nsteps)
    def _prefetch():
        cp(jj, slot).start()

    @pl.when(k == nsteps - 1)
    def _emit():
        out_ref[...] = acc_ref[...]
        s_ref[...] = sacc_ref[...]


def _norm_kernel(acc_ref, s_ref, out_ref):
    d = jnp.sum(s_ref[...].astype(jnp.float32), axis=1, keepdims=True)
    d = jnp.where(d == 0.0, 1.0, d)
    out_ref[...] = acc_ref[...] / d


@jax.jit
def kernel(item_embeds, H):
    item_num, dim = item_embeds.shape
    tag_num = H.shape[1]
    tt = _TT
    nsteps = pl.cdiv(tag_num, tt)

    xa = jnp.concatenate(
        [
            item_embeds.astype(jnp.bfloat16),
            jnp.ones((item_num, 1), jnp.bfloat16),
            jnp.zeros((item_num, 2 * dim - dim - 1), jnp.bfloat16),
        ],
        axis=1,
    )

    acc, s = pl.pallas_call(
        functools.partial(_acc_kernel, nsteps=nsteps, tag_num=tag_num, tt=tt),
        grid=(nsteps,),
        in_specs=[
            pl.BlockSpec(memory_space=pltpu.MemorySpace.HBM),
            pl.BlockSpec((item_num, 2 * dim), lambda k: (0, 0)),
        ],
        out_specs=[
            pl.BlockSpec((item_num, dim), lambda k: (0, 0)),
            pl.BlockSpec((item_num, dim), lambda k: (0, 0)),
        ],
        out_shape=[
            jax.ShapeDtypeStruct((item_num, dim), jnp.float32),
            jax.ShapeDtypeStruct((item_num, dim), jnp.bfloat16),
        ],
        scratch_shapes=[
            pltpu.VMEM((_NBUF, item_num, tt), jnp.float32),
            pltpu.VMEM((item_num, dim), jnp.float32),
            pltpu.VMEM((item_num, dim), jnp.bfloat16),
            pltpu.SemaphoreType.DMA((_NBUF,)),
        ],
        compiler_params=pltpu.CompilerParams(
            dimension_semantics=("arbitrary",),
        ),
    )(H, xa)

    rows = 2000 if item_num % 2000 == 0 else item_num
    return pl.pallas_call(
        _norm_kernel,
        grid=(item_num // rows,),
        in_specs=[
            pl.BlockSpec((rows, dim), lambda i: (i, 0)),
            pl.BlockSpec((rows, dim), lambda i: (i, 0)),
        ],
        out_specs=pl.BlockSpec((rows, dim), lambda i: (i, 0)),
        out_shape=jax.ShapeDtypeStruct((item_num, dim), jnp.float32),
    )(acc, s)
